# Initial kernel scaffold; baseline (speedup 1.0000x reference)
import jax
import jax.numpy as jnp
from jax.experimental import pallas as pl


def kernel(x, edge_index, W1, att_l1, att_r1, b1, W2, att_l2, att_r2, b2):
    raise NotImplementedError("write your pallas kernel here")


# speedup vs baseline: 76.2393x; 76.2393x over previous
import functools
import numpy as np
import jax
import jax.numpy as jnp
from jax import lax
from jax.experimental import pallas as pl
from jax.experimental.pallas import tpu as pltpu
from jax.experimental.pallas import tpu_sc as plsc

HEADS = 8
DIM = 16
NN = 10000
EE = 320000
EPAD = 330240
NW = 32
EPW = EPAD // NW
CH = 80
NCH = EPW // CH
ROW = 144
ACC_ROWS = 10240

_PERM = np.zeros(128, dtype=np.int32)
for _j in range(8):
    for _l in range(16):
        _PERM[16 * _j + _l] = (_l % 8) * 16 + 2 * _j + (_l // 8)

_MEAN = np.zeros((128, 16), dtype=np.float32)
for _i in range(128):
    _MEAN[_i, _PERM[_i] % 16] = 1.0 / 8.0


def _mm_body(x_ref, w_ref, o_ref):
    o_ref[...] = jnp.dot(x_ref[...], w_ref[...], preferred_element_type=jnp.float32)


def _tc_matmul(x, W):
    n, k = x.shape
    m = W.shape[1]
    B = 1000
    return pl.pallas_call(
        _mm_body,
        grid=(n // B,),
        in_specs=[
            pl.BlockSpec((B, k), lambda i: (i, 0)),
            pl.BlockSpec((k, m), lambda i: (0, 0)),
        ],
        out_specs=pl.BlockSpec((B, m), lambda i: (i, 0)),
        out_shape=jax.ShapeDtypeStruct((n, m), jnp.float32),
    )(x, W)


def _combine(acc_ref):
    s = acc_ref[0] + acc_ref[1]
    num = s[:, :128]
    den = s[:, 128:144]
    denb = jnp.concatenate([den] * 8, axis=1)
    return num / (denb + 1e-16)


def _layer2_body(acc_ref, b_ref, w_ref, o_ref):
    h = _combine(acc_ref) + b_ref[...]
    h = jnp.where(h > 0, h, jnp.exp(h) - 1.0)
    o_ref[...] = jnp.dot(h, w_ref[...], preferred_element_type=jnp.float32)


def _tc_layer2(acc, b1p, Wbig2):
    B = 1000
    return pl.pallas_call(
        _layer2_body,
        grid=(NN // B,),
        in_specs=[
            pl.BlockSpec((2, B, ROW), lambda i: (0, i, 0)),
            pl.BlockSpec((1, 128), lambda i: (0, 0)),
            pl.BlockSpec((128, ROW), lambda i: (0, 0)),
        ],
        out_specs=pl.BlockSpec((B, ROW), lambda i: (i, 0)),
        out_shape=jax.ShapeDtypeStruct((NN, ROW), jnp.float32),
    )(acc, b1p, Wbig2)


def _final_body(acc_ref, m_ref, b_ref, o_ref):
    a = _combine(acc_ref)
    o_ref[...] = jnp.dot(a, m_ref[...], preferred_element_type=jnp.float32) + b_ref[...]


def _tc_final(acc, M, b2):
    B = 1000
    return pl.pallas_call(
        _final_body,
        grid=(NN // B,),
        in_specs=[
            pl.BlockSpec((2, B, ROW), lambda i: (0, i, 0)),
            pl.BlockSpec((128, 16), lambda i: (0, 0)),
            pl.BlockSpec((1, 16), lambda i: (0, 0)),
        ],
        out_specs=pl.BlockSpec((B, 16), lambda i: (i, 0)),
        out_shape=jax.ShapeDtypeStruct((NN, 16), jnp.float32),
    )(acc, M, b2)


def _edge_body(hpack_hbm, src_hbm, dst_hbm, zeros_hbm, out_hbm,
               srcv, dstv, Sv, Dv, Ov, accsh, sem1, sem2):
    c = lax.axis_index("c")
    s = lax.axis_index("s")
    wid = s * 2 + c

    zslc = ACC_ROWS // 16
    pltpu.sync_copy(zeros_hbm.at[pl.ds(s * zslc, zslc)],
                    accsh.at[pl.ds(s * zslc, zslc)])
    plsc.subcore_barrier()

    lanes = lax.iota(jnp.int32, 16)
    ridx = lanes ^ 8
    mlow = lanes < 8
    base_w = wid * EPW

    def chunk_body(ci, carry):
        base = base_w + ci * CH
        pltpu.sync_copy(src_hbm.at[pl.ds(base, CH)], srcv)
        pltpu.sync_copy(dst_hbm.at[pl.ds(base, CH)], dstv)

        def remap(j, carry2):
            sv = srcv[pl.ds(j * 16, 16)]
            dv = dstv[pl.ds(j * 16, 16)]
            ids = base + j * 16 + lanes
            m = (sv == dv) & (ids < EE)
            srcv[pl.ds(j * 16, 16)] = jnp.where(m, NN, sv)
            return carry2

        lax.fori_loop(0, CH // 16, remap, 0)

        cp1 = pltpu.async_copy(hpack_hbm.at[srcv], Sv, sem1)
        cp2 = pltpu.async_copy(hpack_hbm.at[dstv], Dv, sem2)
        cp1.wait()
        cp2.wait()

        def ebody(e, carry2):
            sj = [Sv[e, pl.ds(16 * j, 16)] for j in range(8)]
            dj = [Dv[e, pl.ds(16 * j, 16)] for j in range(8)]
            t = sj[0] * dj[0]
            for j in range(1, 8):
                t = t + sj[j] * dj[j]
            logits = t + t.at[ridx].get(mode="promise_in_bounds")
            s8 = Sv[e, pl.ds(128, 16)]
            d8 = Dv[e, pl.ds(128, 16)]
            b = jnp.where(mlow, s8, d8)
            basef = b + b.at[ridx].get(mode="promise_in_bounds")
            sig = 1.0 / (1.0 + jnp.exp(-logits))
            alpha = basef * sig
            alpha = jnp.where(alpha >= 0, alpha, 0.2 * alpha)
            ex = jnp.exp(alpha)
            for j in range(8):
                Ov[e, pl.ds(16 * j, 16)] = sj[j] * ex
            Ov[e, pl.ds(128, 16)] = ex
            return carry2

        lax.fori_loop(0, CH, ebody, 0)

        pltpu.sync_copy(Ov, accsh.at[dstv], add=True)
        return carry

    lax.fori_loop(0, NCH, chunk_body, 0)
    plsc.subcore_barrier()

    oslc = ACC_ROWS // 16
    pltpu.sync_copy(accsh.at[pl.ds(s * oslc, oslc)],
                    out_hbm.at[c, pl.ds(s * oslc, oslc)])


_edge_pass = functools.partial(
    pl.kernel,
    out_type=jax.ShapeDtypeStruct((2, ACC_ROWS, ROW), jnp.float32),
    mesh=plsc.VectorSubcoreMesh(core_axis_name="c", subcore_axis_name="s"),
    compiler_params=pltpu.CompilerParams(use_tc_tiling_on_sc=False),
    scratch_types=[
        pltpu.VMEM((CH,), jnp.int32),
        pltpu.VMEM((CH,), jnp.int32),
        pltpu.VMEM((CH, ROW), jnp.float32),
        pltpu.VMEM((CH, ROW), jnp.float32),
        pltpu.VMEM((CH, ROW), jnp.float32),
        pltpu.VMEM_SHARED((ACC_ROWS, ROW), jnp.float32),
        pltpu.SemaphoreType.DMA,
        pltpu.SemaphoreType.DMA,
    ],
)(_edge_body)


def _prep_weights(W, att_l, att_r, permute_rows):
    Al = (W.reshape(-1, HEADS, DIM) * att_l[0][None]).sum(-1)
    Ar = (W.reshape(-1, HEADS, DIM) * att_r[0][None]).sum(-1)
    Wbig = jnp.concatenate([W[:, _PERM], Al, Ar], axis=1)
    if permute_rows:
        Wbig = Wbig[_PERM, :]
    return Wbig


def kernel(x, edge_index, W1, att_l1, att_r1, b1, W2, att_l2, att_r2, b2):
    src, dst = edge_index[0], edge_index[1]
    loop = jnp.arange(NN, dtype=src.dtype)
    npad = EPAD - EE - NN
    padv = jnp.full((npad,), NN, src.dtype)
    src2 = jnp.concatenate([src, loop, padv])
    dst2 = jnp.concatenate([dst, loop, padv])

    Wbig1 = _prep_weights(W1, att_l1, att_r1, permute_rows=False)
    Wbig2 = _prep_weights(W2, att_l2, att_r2, permute_rows=True)
    b1p = b1[_PERM][None]
    dummy = jnp.concatenate(
        [jnp.zeros(128), jnp.full((8,), -1e30), jnp.zeros(8)]
    ).astype(jnp.float32)[None]
    zeros = jnp.zeros((ACC_ROWS, ROW), jnp.float32)
    M = jnp.asarray(_MEAN)

    Hp1 = jnp.concatenate([_tc_matmul(x, Wbig1), dummy], axis=0)
    acc1 = _edge_pass(Hp1, src2, dst2, zeros)[:, :NN, :]
    Hp2 = jnp.concatenate([_tc_layer2(acc1, b1p, Wbig2), dummy], axis=0)
    acc2 = _edge_pass(Hp2, src2, dst2, zeros)[:, :NN, :]
    return _tc_final(acc2, M, b2[None])
